# baseline (speedup 1.0000x reference)
import jax
import jax.numpy as jnp
from jax import lax
from jax.experimental import pallas as pl
from jax.experimental.pallas import tpu as pltpu
from jax.experimental.pallas import tpu_sc as plsc

D = 32
LANES = 16
NC = 2
NS = 16
NW = NC * NS
B = 16384
V = 1000000
BLK = 1024
NFULL = V // BLK
TAILV = NFULL * BLK
CAP = 1280
NSLOT = NW * CAP
RING = 256
SVPAD = 2 * B + CAP + 32


def _worker_id():
    return lax.axis_index("s") * NC + lax.axis_index("c")


def _block_start(wid):
    return jnp.where(wid < 16, 31 * wid, 496 + 30 * (wid - 16))


def _extract_body(sv_hbm, st_hbm, embT_hbm, tail_hbm, x_hbm,
                   sv_s, st_s, dbuf, stag, sem, sem2):
    wid = _worker_id()
    pltpu.sync_copy(st_hbm, st_s)
    stv = st_s[pl.ds(wid, LANES)]
    s0 = stv[0]
    n = jnp.minimum(stv[1] - s0, CAP)
    a0 = s0 & 7
    start_al = pl.multiple_of(s0 - a0, 8)
    pltpu.sync_copy(sv_hbm.at[pl.ds(start_al, CAP + 32)], sv_s)

    b0 = _block_start(wid)
    nb = jnp.where(wid < 16, 31, 30)
    is_last = wid == NW - 1

    def fire(bi):
        blk = b0 + bi
        return pltpu.async_copy(
            embT_hbm.at[:, pl.ds(blk * BLK, BLK)], dbuf.at[bi & 1], sem)

    lane_iota = lax.broadcasted_iota(jnp.int32, (LANES,), 0)

    def extract_run(carry, base, limit, buf):
        e0, f0 = carry

        def cond(c):
            e1, _ = c
            v = sv_s[pl.ds(a0 + e1, LANES)][0]
            return (e1 < n) & (v < limit)

        def body(c):
            e1, f1 = c
            v = sv_s[pl.ds(a0 + e1, LANES)][0]
            cc = v - base
            b_splat = jnp.zeros((LANES,), jnp.int32) + buf
            c_splat = jnp.zeros((LANES,), jnp.int32) + cc
            row = e1 & (RING - 1)
            g0 = plsc.load_gather(dbuf, [b_splat, lane_iota, c_splat])
            g1 = plsc.load_gather(dbuf, [b_splat, lane_iota + LANES, c_splat])
            stag[row, pl.ds(0, LANES)] = g0
            stag[row, pl.ds(LANES, LANES)] = g1
            return e1 + 1, f1

        e1, f1 = lax.while_loop(cond, body, (e0, f0))

        def flush_one(c):
            def do_flush(c2):
                e2, f2 = c2
                pltpu.async_copy(
                    stag.at[pl.ds(pl.multiple_of(f2 & (RING - 1), 128), 128)],
                    x_hbm.at[pl.ds(pl.multiple_of(wid * CAP + f2, 128), 128)],
                    sem2)
                return e2, f2 + 128

            return lax.cond(c[0] - c[1] >= 128, do_flush, lambda c2: c2, c)

        return flush_one(flush_one((e1, f1)))

    fire(0)

    def block_step(bi, carry):
        @pl.when(bi + 1 < nb)
        def _():
            fire(bi + 1)

        blk = b0 + bi
        pltpu.make_async_copy(
            embT_hbm.at[:, pl.ds(blk * BLK, BLK)], dbuf.at[bi & 1], sem
        ).wait()
        return extract_run(carry, blk * BLK, (blk + 1) * BLK, bi & 1)

    carry = lax.fori_loop(0, nb, block_step, (jnp.int32(0), jnp.int32(0)))

    @pl.when(is_last)
    def _():
        pltpu.sync_copy(tail_hbm, dbuf.at[0])

    def tail_or_not(c):
        return lax.cond(
            is_last,
            lambda cc: extract_run(cc, jnp.int32(TAILV), jnp.int32(V + 1), 0),
            lambda cc: cc,
            c)

    e, f = tail_or_not(carry)

    def final_flush(t, c):
        e1, f1 = c

        def do_flush(c2):
            e2, f2 = c2
            pltpu.async_copy(
                stag.at[pl.ds(pl.multiple_of(f2 & (RING - 1), 128), 128)],
                x_hbm.at[pl.ds(pl.multiple_of(wid * CAP + f2, 128), 128)],
                sem2)
            return e2, f2 + 128

        return lax.cond(f1 < e1, do_flush, lambda c2: c2, c)

    e, f = lax.fori_loop(0, CAP // 128, final_flush, (e, f))

    def drain(t, c):
        @pl.when(t * 128 < f)
        def _():
            pltpu.make_async_copy(
                stag.at[pl.ds(0, 128)],
                x_hbm.at[pl.ds(wid * CAP, 128)], sem2).wait()
        return c

    lax.fori_loop(0, CAP // 128, drain, 0)


GUARD = 0x7FFF0000


def _slot_body(order_hbm, sv_hbm, slot_hbm, st_out, ord_v, sv_v, st_v, slot_v):
    wid = _worker_id()

    @pl.when(wid == 0)
    def _():
        pltpu.sync_copy(order_hbm, ord_v)
        pltpu.sync_copy(sv_hbm, sv_v)
        lane_iota = lax.broadcasted_iota(jnp.int32, (LANES,), 0)
        lane0 = lane_iota == 0

        for q in range(3):
            st_v[pl.ds(q * LANES, LANES)] = jnp.full((LANES,), GUARD, jnp.int32)

        def wstart(w):
            return jnp.where(w < 16, 31 * w, 496 + 30 * (w - 16)) * BLK

        def chunk_a(c, w):
            v = sv_v[pl.ds(c * LANES, LANES)]
            v_last = v[LANES - 1]

            def acond(ww):
                return (ww < NW) & (wstart(ww) <= v_last)

            def abody(ww):
                ws = wstart(ww)
                less = lax.reduce_sum((v < ws).astype(jnp.int32), (0,))
                rho = c * LANES + less
                plsc.store_scatter(st_v, [jnp.zeros((LANES,), jnp.int32) + ww],
                                   jnp.zeros((LANES,), jnp.int32) + rho,
                                   mask=lane0)
                return ww + 1

            return lax.while_loop(acond, abody, w)

        w_end = lax.fori_loop(0, (2 * B) // LANES, chunk_a, jnp.int32(0))

        def fill(ww):
            plsc.store_scatter(st_v, [jnp.zeros((LANES,), jnp.int32) + ww],
                               jnp.full((LANES,), 2 * B, jnp.int32), mask=lane0)
            return ww + 1

        lax.while_loop(lambda ww: ww <= NW, fill, w_end)

        def chunk_b(c, w):
            rho = lane_iota + c * LANES
            tgt = ord_v[pl.ds(c * LANES, LANES)]
            stv = st_v[pl.ds(w, LANES)]
            s_w = stv[0]
            s_w1 = stv[1]
            hi = rho >= s_w1
            ow = w + hi.astype(jnp.int32)
            base = jnp.where(hi, s_w1, s_w)
            slot_vec = ow * CAP + rho - base
            plsc.store_scatter(slot_v, [tgt], slot_vec)

            def wcond(ww):
                return st_v[pl.ds(ww + 1, LANES)][0] <= (c + 1) * LANES

            return lax.while_loop(wcond, lambda ww: ww + 1, w)

        lax.fori_loop(0, (2 * B) // LANES, chunk_b, jnp.int32(0))
        pltpu.sync_copy(slot_v, slot_hbm)
        pltpu.sync_copy(st_v, st_out)


def _dot_body(us_hbm, vs_hbm, de_hbm, x_hbm, out_hbm,
              us_v, vs_v, de_v, gu, gv, dot_v, o_v, sem):
    wid = _worker_id()
    nch = 4
    base = wid * nch
    pltpu.sync_copy(us_hbm.at[pl.ds(base, nch)], us_v)
    pltpu.sync_copy(vs_hbm.at[pl.ds(base, nch)], vs_v)
    pltpu.sync_copy(de_hbm.at[pl.ds(base, nch)], de_v)

    def fire(k):
        b = k & 1
        cu = pltpu.async_copy(x_hbm.at[us_v.at[k]], gu.at[b], sem)
        cv = pltpu.async_copy(x_hbm.at[vs_v.at[k]], gv.at[b], sem)
        return cu, cv

    lane_iota = lax.broadcasted_iota(jnp.int32, (LANES,), 0)
    last_lane = lane_iota == (LANES - 1)

    pending = fire(0)
    for k in range(nch):
        nxt = fire(k + 1) if k + 1 < nch else None
        for c in pending:
            c.wait()
        b = k & 1

        def pair_dot(p, _, b=b, k=k):
            u0 = gu[b, p, pl.ds(0, LANES)]
            u1 = gu[b, p, pl.ds(LANES, LANES)]
            v0 = gv[b, p, pl.ds(0, LANES)]
            v1 = gv[b, p, pl.ds(LANES, LANES)]
            s = u0 * v0 + u1 * v1
            cs = plsc.cumsum(s)
            idx = jnp.zeros((LANES,), jnp.int32) + (k * 128 + p)
            plsc.store_scatter(dot_v, [idx], cs, mask=last_lane)
            return _

        lax.fori_loop(0, 128, pair_dot, 0)
        pending = nxt

    def epilogue(g, _):
        dev = de_v[g // 8, pl.ds((g % 8) * LANES, LANES)]
        dd = dot_v[pl.ds(g * LANES, LANES)]
        x = dev * dd
        s = 1.0 / (1.0 + jnp.exp(-x))
        o_v[pl.ds(g * LANES, LANES)] = -s
        return _

    lax.fori_loop(0, (B // NW) // LANES, epilogue, 0)

    pltpu.sync_copy(o_v, out_hbm.at[pl.ds(wid * (B // NW), B // NW)])


def kernel(pair, emb):
    i = pair[:, 0].astype(jnp.int32)
    j = pair[:, 1].astype(jnp.int32)
    de = pair[:, 2].astype(jnp.float32).reshape(B // 128, 128)
    embT = emb.T

    vv = jnp.concatenate([i, j])
    sv = jnp.sort(vv)
    order = jnp.argsort(vv).astype(jnp.int32)

    sv_pad = jnp.pad(sv, (0, SVPAD - 2 * B), constant_values=V + 10)
    tailp = jnp.pad(embT[:, TAILV:], ((0, 0), (0, BLK - (V - TAILV))))

    mesh = plsc.VectorSubcoreMesh(core_axis_name="c", subcore_axis_name="s")

    kslot = pl.kernel(
        _slot_body,
        out_type=(jax.ShapeDtypeStruct((2 * B,), jnp.int32),
                  jax.ShapeDtypeStruct((48,), jnp.int32)),
        mesh=mesh,
        compiler_params=pltpu.CompilerParams(needs_layout_passes=False),
        scratch_types=[
            pltpu.VMEM((2 * B,), jnp.int32),
            pltpu.VMEM((2 * B,), jnp.int32),
            pltpu.VMEM((48,), jnp.int32),
            pltpu.VMEM((2 * B,), jnp.int32),
        ],
    )
    slot, st48 = kslot(order, sv)
    uslot = slot[:B].reshape(B // 128, 128)
    vslot = slot[B:].reshape(B // 128, 128)

    k1 = pl.kernel(
        _extract_body,
        out_type=jax.ShapeDtypeStruct((NSLOT, 128), jnp.float32),
        mesh=mesh,
        compiler_params=pltpu.CompilerParams(needs_layout_passes=False),
        scratch_types=[
            pltpu.VMEM((CAP + 32,), jnp.int32),
            pltpu.VMEM((48,), jnp.int32),
            pltpu.VMEM((2, D, BLK), jnp.float32),
            pltpu.VMEM((RING, 128), jnp.float32),
            pltpu.SemaphoreType.DMA,
            pltpu.SemaphoreType.DMA,
        ],
    )
    xrows = k1(sv_pad, st48, embT, tailp)

    k2 = pl.kernel(
        _dot_body,
        out_type=jax.ShapeDtypeStruct((B,), jnp.float32),
        mesh=mesh,
        compiler_params=pltpu.CompilerParams(needs_layout_passes=False),
        scratch_types=[
            pltpu.VMEM((4, 128), jnp.int32),
            pltpu.VMEM((4, 128), jnp.int32),
            pltpu.VMEM((4, 128), jnp.float32),
            pltpu.VMEM((2, 128, 128), jnp.float32),
            pltpu.VMEM((2, 128, 128), jnp.float32),
            pltpu.VMEM((B // NW,), jnp.float32),
            pltpu.VMEM((B // NW,), jnp.float32),
            pltpu.SemaphoreType.DMA,
        ],
    )
    out = k2(uslot, vslot, de, xrows)
    return out.reshape(B, 1)


# speedup vs baseline: 1.1549x; 1.1549x over previous
import jax
import jax.numpy as jnp
from jax import lax
from jax.experimental import pallas as pl
from jax.experimental.pallas import tpu as pltpu
from jax.experimental.pallas import tpu_sc as plsc

D = 32
LANES = 16
NC = 2
NS = 16
NW = NC * NS
B = 16384
V = 1000000
BLK = 1024
NFULL = V // BLK
TAILV = NFULL * BLK
CAP = 1280
NSLOT = NW * CAP
RING = 256
SVPAD = 2 * B + CAP + 32


def _worker_id():
    return lax.axis_index("s") * NC + lax.axis_index("c")


def _block_start(wid):
    return jnp.where(wid < 16, 31 * wid, 496 + 30 * (wid - 16))


def _extract_body(sv_hbm, st_hbm, embT_hbm, tail_hbm, x_hbm,
                   sv_s, st_s, dbuf, stag, sem, sem2):
    wid = _worker_id()
    pltpu.sync_copy(st_hbm, st_s)
    stv = st_s[pl.ds(wid, LANES)]
    s0 = stv[0]
    n = jnp.minimum(stv[1] - s0, CAP)
    a0 = s0 & 7
    start_al = pl.multiple_of(s0 - a0, 8)
    pltpu.sync_copy(sv_hbm.at[pl.ds(start_al, CAP + 32)], sv_s)

    b0 = _block_start(wid)
    nb = jnp.where(wid < 16, 31, 30)
    is_last = wid == NW - 1

    def fire(bi):
        blk = b0 + bi
        return pltpu.async_copy(
            embT_hbm.at[:, pl.ds(blk * BLK, BLK)], dbuf.at[bi & 1], sem)

    lane_iota = lax.broadcasted_iota(jnp.int32, (LANES,), 0)

    def extract_run(carry, base, limit, buf):
        e0, f0 = carry

        def cond(c):
            e1, _ = c
            v = sv_s[pl.ds(a0 + e1, LANES)][0]
            return (e1 < n) & (v < limit)

        def body(c):
            e1, f1 = c
            v = sv_s[pl.ds(a0 + e1, LANES)][0]
            cc = v - base
            b_splat = jnp.zeros((LANES,), jnp.int32) + buf
            c_splat = jnp.zeros((LANES,), jnp.int32) + cc
            row = e1 & (RING - 1)
            g0 = plsc.load_gather(dbuf, [b_splat, lane_iota, c_splat])
            g1 = plsc.load_gather(dbuf, [b_splat, lane_iota + LANES, c_splat])
            stag[row, pl.ds(0, LANES)] = g0
            stag[row, pl.ds(LANES, LANES)] = g1
            return e1 + 1, f1

        e1, f1 = lax.while_loop(cond, body, (e0, f0))

        def flush_one(c):
            def do_flush(c2):
                e2, f2 = c2
                pltpu.async_copy(
                    stag.at[pl.ds(pl.multiple_of(f2 & (RING - 1), 128), 128)],
                    x_hbm.at[pl.ds(pl.multiple_of(wid * CAP + f2, 128), 128)],
                    sem2)
                return e2, f2 + 128

            return lax.cond(c[0] - c[1] >= 128, do_flush, lambda c2: c2, c)

        return flush_one(flush_one((e1, f1)))

    fire(0)

    def block_step(bi, carry):
        @pl.when(bi + 1 < nb)
        def _():
            fire(bi + 1)

        blk = b0 + bi
        pltpu.make_async_copy(
            embT_hbm.at[:, pl.ds(blk * BLK, BLK)], dbuf.at[bi & 1], sem
        ).wait()
        return extract_run(carry, blk * BLK, (blk + 1) * BLK, bi & 1)

    carry = lax.fori_loop(0, nb, block_step, (jnp.int32(0), jnp.int32(0)))

    @pl.when(is_last)
    def _():
        pltpu.sync_copy(tail_hbm, dbuf.at[0])

    def tail_or_not(c):
        return lax.cond(
            is_last,
            lambda cc: extract_run(cc, jnp.int32(TAILV), jnp.int32(V + 1), 0),
            lambda cc: cc,
            c)

    e, f = tail_or_not(carry)

    def final_flush(t, c):
        e1, f1 = c

        def do_flush(c2):
            e2, f2 = c2
            pltpu.async_copy(
                stag.at[pl.ds(pl.multiple_of(f2 & (RING - 1), 128), 128)],
                x_hbm.at[pl.ds(pl.multiple_of(wid * CAP + f2, 128), 128)],
                sem2)
            return e2, f2 + 128

        return lax.cond(f1 < e1, do_flush, lambda c2: c2, c)

    e, f = lax.fori_loop(0, CAP // 128, final_flush, (e, f))

    def drain(t, c):
        @pl.when(t * 128 < f)
        def _():
            pltpu.make_async_copy(
                stag.at[pl.ds(0, 128)],
                x_hbm.at[pl.ds(wid * CAP, 128)], sem2).wait()
        return c

    lax.fori_loop(0, CAP // 128, drain, 0)


def _slot_body(order_hbm, st_hbm, slot_hbm, ord_v, st_v, slot_v):
    wid = _worker_id()

    @pl.when(wid == 0)
    def _():
        pltpu.sync_copy(order_hbm, ord_v)
        pltpu.sync_copy(st_hbm, st_v)
        lane_iota = lax.broadcasted_iota(jnp.int32, (LANES,), 0)

        def chunk_b(c, w):
            rho = lane_iota + c * LANES
            tgt = ord_v[pl.ds(c * LANES, LANES)]
            stv = st_v[pl.ds(w, LANES)]
            s_w = stv[0]
            s_w1 = stv[1]
            hi = rho >= s_w1
            ow = w + hi.astype(jnp.int32)
            base = jnp.where(hi, s_w1, s_w)
            slot_vec = ow * CAP + rho - base
            plsc.store_scatter(slot_v, [tgt], slot_vec)

            def wcond(ww):
                return st_v[pl.ds(ww + 1, LANES)][0] <= (c + 1) * LANES

            return lax.while_loop(wcond, lambda ww: ww + 1, w)

        lax.fori_loop(0, (2 * B) // LANES, chunk_b, jnp.int32(0))
        pltpu.sync_copy(slot_v, slot_hbm)


def _dot_body(us_hbm, vs_hbm, de_hbm, x_hbm, out_hbm,
              us_v, vs_v, de_v, gu, gv, dot_v, o_v, sem):
    wid = _worker_id()
    nch = 4
    base = wid * nch
    pltpu.sync_copy(us_hbm.at[pl.ds(base, nch)], us_v)
    pltpu.sync_copy(vs_hbm.at[pl.ds(base, nch)], vs_v)
    pltpu.sync_copy(de_hbm.at[pl.ds(base, nch)], de_v)

    def fire(k):
        b = k & 1
        cu = pltpu.async_copy(x_hbm.at[us_v.at[k]], gu.at[b], sem)
        cv = pltpu.async_copy(x_hbm.at[vs_v.at[k]], gv.at[b], sem)
        return cu, cv

    lane_iota = lax.broadcasted_iota(jnp.int32, (LANES,), 0)
    last_lane = lane_iota == (LANES - 1)

    pending = fire(0)
    for k in range(nch):
        nxt = fire(k + 1) if k + 1 < nch else None
        for c in pending:
            c.wait()
        b = k & 1

        def pair_dot(p, _, b=b, k=k):
            u0 = gu[b, p, pl.ds(0, LANES)]
            u1 = gu[b, p, pl.ds(LANES, LANES)]
            v0 = gv[b, p, pl.ds(0, LANES)]
            v1 = gv[b, p, pl.ds(LANES, LANES)]
            s = u0 * v0 + u1 * v1
            cs = plsc.cumsum(s)
            idx = jnp.zeros((LANES,), jnp.int32) + (k * 128 + p)
            plsc.store_scatter(dot_v, [idx], cs, mask=last_lane)
            return _

        lax.fori_loop(0, 128, pair_dot, 0)
        pending = nxt

    def epilogue(g, _):
        dev = de_v[g // 8, pl.ds((g % 8) * LANES, LANES)]
        dd = dot_v[pl.ds(g * LANES, LANES)]
        x = dev * dd
        s = 1.0 / (1.0 + jnp.exp(-x))
        o_v[pl.ds(g * LANES, LANES)] = -s
        return _

    lax.fori_loop(0, (B // NW) // LANES, epilogue, 0)

    pltpu.sync_copy(o_v, out_hbm.at[pl.ds(wid * (B // NW), B // NW)])


def kernel(pair, emb):
    i = pair[:, 0].astype(jnp.int32)
    j = pair[:, 1].astype(jnp.int32)
    de = pair[:, 2].astype(jnp.float32).reshape(B // 128, 128)
    embT = emb.T

    vv = jnp.concatenate([i, j])
    sv = jnp.sort(vv)
    order = jnp.argsort(vv).astype(jnp.int32)

    bs_py = [31 * w if w < 16 else 496 + 30 * (w - 16) for w in range(NW)]
    wstart_v = jnp.array([b * BLK for b in bs_py], jnp.int32)
    starts = jnp.searchsorted(sv, wstart_v).astype(jnp.int32)
    starts33 = jnp.concatenate([starts, jnp.array([2 * B], jnp.int32)])
    st48 = jnp.pad(starts33, (0, 48 - 33), constant_values=0x7FFF0000)

    sv_pad = jnp.pad(sv, (0, SVPAD - 2 * B), constant_values=V + 10)
    tailp = jnp.pad(embT[:, TAILV:], ((0, 0), (0, BLK - (V - TAILV))))

    mesh = plsc.VectorSubcoreMesh(core_axis_name="c", subcore_axis_name="s")

    kslot = pl.kernel(
        _slot_body,
        out_type=jax.ShapeDtypeStruct((2 * B,), jnp.int32),
        mesh=mesh,
        compiler_params=pltpu.CompilerParams(needs_layout_passes=False),
        scratch_types=[
            pltpu.VMEM((2 * B,), jnp.int32),
            pltpu.VMEM((48,), jnp.int32),
            pltpu.VMEM((2 * B,), jnp.int32),
        ],
    )
    slot = kslot(order, st48)
    uslot = slot[:B].reshape(B // 128, 128)
    vslot = slot[B:].reshape(B // 128, 128)

    k1 = pl.kernel(
        _extract_body,
        out_type=jax.ShapeDtypeStruct((NSLOT, 128), jnp.float32),
        mesh=mesh,
        compiler_params=pltpu.CompilerParams(needs_layout_passes=False),
        scratch_types=[
            pltpu.VMEM((CAP + 32,), jnp.int32),
            pltpu.VMEM((48,), jnp.int32),
            pltpu.VMEM((2, D, BLK), jnp.float32),
            pltpu.VMEM((RING, 128), jnp.float32),
            pltpu.SemaphoreType.DMA,
            pltpu.SemaphoreType.DMA,
        ],
    )
    xrows = k1(sv_pad, st48, embT, tailp)

    k2 = pl.kernel(
        _dot_body,
        out_type=jax.ShapeDtypeStruct((B,), jnp.float32),
        mesh=mesh,
        compiler_params=pltpu.CompilerParams(needs_layout_passes=False),
        scratch_types=[
            pltpu.VMEM((4, 128), jnp.int32),
            pltpu.VMEM((4, 128), jnp.int32),
            pltpu.VMEM((4, 128), jnp.float32),
            pltpu.VMEM((2, 128, 128), jnp.float32),
            pltpu.VMEM((2, 128, 128), jnp.float32),
            pltpu.VMEM((B // NW,), jnp.float32),
            pltpu.VMEM((B // NW,), jnp.float32),
            pltpu.SemaphoreType.DMA,
        ],
    )
    out = k2(uslot, vslot, de, xrows)
    return out.reshape(B, 1)
